# baseline (speedup 1.0000x reference)
import functools

import jax
import jax.numpy as jnp
from jax import lax
from jax.experimental import pallas as pl
from jax.experimental.pallas import tpu as pltpu
from jax.experimental.pallas import tpu_sc as plsc

NN = 10000
NE = 320000
H = 128

NC = 2
NS = 16
NW = NC * NS
EPW = NE // NW
CH = 80
NCHUNK = EPW // CH

NSLOT = 5
PF = 3

SCH = 40
SNCHUNK = EPW // SCH
NSLICE = 5
NACC = NN // SCH
NZCH = -(-NACC // NS)

_mesh = plsc.VectorSubcoreMesh(core_axis_name="c", subcore_axis_name="s")


def _make_gather(epw, nchunk):
    @functools.partial(
        pl.kernel,
        out_type=(
            jax.ShapeDtypeStruct((epw * NW, H), jnp.float32),
            jax.ShapeDtypeStruct((epw * NW, H), jnp.float32),
        ),
        mesh=_mesh,
        scratch_types=[
            pltpu.VMEM((epw,), jnp.int32),
            pltpu.VMEM((NSLOT, CH, H), jnp.float32),
            pltpu.VMEM((epw,), jnp.int32),
            pltpu.VMEM((NSLOT, CH, H), jnp.float32),
            pltpu.SemaphoreType.DMA((2, NSLOT)),
            pltpu.SemaphoreType.DMA((2, NSLOT)),
        ],
    )
    def gather(x_hbm, src_hbm, dst_hbm, s_out, r_out,
               idx_s, rows_s, idx_r, rows_r, gsem, osem):
        wid = lax.axis_index("c") * NS + lax.axis_index("s")
        base = wid * epw

        pltpu.sync_copy(src_hbm.at[pl.ds(base, epw)], idx_s)
        pltpu.sync_copy(dst_hbm.at[pl.ds(base, epw)], idx_r)

        def start(jp, sp):
            iofs = jp * CH
            pltpu.async_copy(x_hbm.at[idx_s.at[pl.ds(iofs, CH)]],
                             rows_s.at[sp], gsem.at[0, sp])
            pltpu.async_copy(x_hbm.at[idx_r.at[pl.ds(iofs, CH)]],
                             rows_r.at[sp], gsem.at[1, sp])

        def wait_outs(sp):
            pltpu.make_async_copy(
                rows_s.at[sp], s_out.at[pl.ds(base, CH)], osem.at[0, sp]).wait()
            pltpu.make_async_copy(
                rows_r.at[sp], r_out.at[pl.ds(base, CH)], osem.at[1, sp]).wait()

        for p in range(PF):
            start(p, p)

        def group(gg, carry):
            for s in range(NSLOT):
                j = gg * NSLOT + s
                sp = (s + PF) % NSLOT
                jp = j + PF

                @pl.when(jp < nchunk)
                def _():
                    @pl.when(jp >= NSLOT)
                    def _():
                        wait_outs(sp)

                    start(jp, sp)

                pltpu.make_async_copy(
                    x_hbm.at[idx_s.at[pl.ds(0, CH)]], rows_s.at[s],
                    gsem.at[0, s]).wait()
                pltpu.make_async_copy(
                    x_hbm.at[idx_r.at[pl.ds(0, CH)]], rows_r.at[s],
                    gsem.at[1, s]).wait()
                off = base + j * CH
                pltpu.async_copy(rows_s.at[s], s_out.at[pl.ds(off, CH)],
                                 osem.at[0, s])
                pltpu.async_copy(rows_r.at[s], r_out.at[pl.ds(off, CH)],
                                 osem.at[1, s])
            return carry

        lax.fori_loop(0, nchunk // NSLOT, group, 0)
        for s in range(NSLOT):
            wait_outs(s)

    return gather


def _make_scatter(epw, snchunk, edge0):
    @functools.partial(
        pl.kernel,
        out_type=jax.ShapeDtypeStruct((NC * NN, H), jnp.float32),
        mesh=_mesh,
        scratch_types=[
            pltpu.VMEM((NSLOT, SCH), jnp.int32),
            pltpu.VMEM((NSLOT, SCH, H), jnp.float32),
            pltpu.VMEM_SHARED((NN, H), jnp.float32),
            pltpu.SemaphoreType.DMA((NSLOT,)),
            pltpu.SemaphoreType.DMA((NSLOT,)),
        ],
    )
    def scatter(e_hbm, dst_hbm, init_hbm, out_hbm, idx_v, rows_v, acc_sh,
                lsem, asem):
        cid = lax.axis_index("c")
        sid = lax.axis_index("s")
        base = edge0 + (cid * NS + sid) * epw

        for t in range(NZCH):
            k = t * NS + sid

            @pl.when(k < NACC)
            def _():
                pltpu.sync_copy(init_hbm.at[pl.ds(cid * NN + k * SCH, SCH)],
                                rows_v.at[0])
                pltpu.sync_copy(rows_v.at[0], acc_sh.at[pl.ds(k * SCH, SCH)])

        plsc.subcore_barrier()

        def startload(jp, sp):
            off = base + jp * SCH
            pltpu.sync_copy(dst_hbm.at[pl.ds(off, SCH)], idx_v.at[sp])
            pltpu.async_copy(e_hbm.at[pl.ds(off, SCH)], rows_v.at[sp],
                             lsem.at[sp])

        def wait_add(sp):
            pltpu.make_async_copy(
                rows_v.at[sp], acc_sh.at[pl.ds(0, SCH)], asem.at[sp]).wait()

        for p in range(PF):
            startload(p, p)

        def group(gg, carry):
            for s in range(NSLOT):
                j = gg * NSLOT + s
                sp = (s + PF) % NSLOT
                jp = j + PF

                @pl.when(jp < snchunk)
                def _():
                    @pl.when(jp >= NSLOT)
                    def _():
                        wait_add(sp)

                    startload(jp, sp)

                pltpu.make_async_copy(
                    e_hbm.at[pl.ds(base, SCH)], rows_v.at[s],
                    lsem.at[s]).wait()
                pltpu.async_copy(rows_v.at[s], acc_sh.at[idx_v.at[s]],
                                 asem.at[s], add=True)
            return carry

        lax.fori_loop(0, snchunk // NSLOT, group, 0)
        for s in range(NSLOT):
            wait_add(s)
        plsc.subcore_barrier()

        for t in range(NZCH):
            k = t * NS + sid

            @pl.when(k < NACC)
            def _():
                pltpu.sync_copy(acc_sh.at[pl.ds(k * SCH, SCH)], rows_v.at[0])
                pltpu.sync_copy(rows_v.at[0],
                                out_hbm.at[pl.ds(cid * NN + k * SCH, SCH)])

    return scatter


BE = 2560


def _ln_relu(z, g, bt):
    h = jnp.maximum(z, 0.0)
    mu = jnp.mean(h, axis=-1, keepdims=True)
    var = jnp.mean((h - mu) ** 2, axis=-1, keepdims=True)
    return (h - mu) * lax.rsqrt(var + 1e-5) * g + bt


def _mxu_ln_relu(z, g, bt, jm):
    h = jnp.maximum(z, 0.0)
    hb = h.astype(jnp.bfloat16)
    mu = jnp.dot(hb, jm, preferred_element_type=jnp.float32)
    q = hb * hb
    ms = jnp.dot(q, jm, preferred_element_type=jnp.float32)
    inv = lax.rsqrt(ms - mu * mu + 1e-5)
    return ((h - mu) * inv * g + bt).astype(jnp.bfloat16)


def _edge_mlp_body(ea_ref, s_ref, r_ref, wa_ref, ws_ref, wr_ref,
                   wk_ref, bk_ref, out_ref):
    bks = bk_ref[...]
    wks = wk_ref[...]
    jm = jnp.full((H, H), 1.0 / H, dtype=jnp.bfloat16)
    s_bf = s_ref[...].astype(jnp.bfloat16)
    r_bf = r_ref[...].astype(jnp.bfloat16)
    z = (jnp.dot(ea_ref[...], wa_ref[...], preferred_element_type=jnp.float32)
         + jnp.dot(s_bf, ws_ref[...], preferred_element_type=jnp.float32)
         + jnp.dot(r_bf, wr_ref[...], preferred_element_type=jnp.float32)
         + bks[0:1])
    h = _mxu_ln_relu(z, bks[1:2], bks[2:3], jm)
    h = _mxu_ln_relu(jnp.dot(h, wks[0], preferred_element_type=jnp.float32)
                     + bks[3:4], bks[4:5], bks[5:6], jm)
    h = _mxu_ln_relu(jnp.dot(h, wks[1], preferred_element_type=jnp.float32)
                     + bks[6:7], bks[7:8], bks[8:9], jm)
    out_ref[...] = (jnp.dot(h, wks[2], preferred_element_type=jnp.float32)
                    + bks[9:10])


def _edge_mlp(block_off, prev_buf, ea_p, sender, receiver, wa_p, ws, wr,
              wstk, bstk):
    grid = (sender.shape[0] // BE,)
    body = _edge_mlp_body
    if prev_buf is not None:
        body = lambda prev_ref, *a: _edge_mlp_body(*a)
    in_specs = [
            pl.BlockSpec((BE, 8), lambda i: (i + block_off, 0)),
            pl.BlockSpec((BE, H), lambda i: (i, 0)),
            pl.BlockSpec((BE, H), lambda i: (i, 0)),
            pl.BlockSpec((8, H), lambda i: (0, 0)),
            pl.BlockSpec((H, H), lambda i: (0, 0)),
            pl.BlockSpec((H, H), lambda i: (0, 0)),
            pl.BlockSpec((3, H, H), lambda i: (0, 0, 0)),
            pl.BlockSpec((10, H), lambda i: (0, 0)),
    ]
    args = (ea_p, sender, receiver, wa_p, ws, wr, wstk, bstk)
    aliases = {}
    if prev_buf is not None:
        in_specs = [pl.BlockSpec(memory_space=pl.ANY)] + in_specs
        args = (prev_buf,) + args
        aliases = {0: 0}
    return pl.pallas_call(
        body,
        grid=grid,
        in_specs=in_specs,
        out_specs=pl.BlockSpec((BE, H), lambda i: (i + block_off, 0)),
        out_shape=jax.ShapeDtypeStruct((NE, H), jnp.float32),
        input_output_aliases=aliases,
    )(*args)


BN = 2000


def _node_mlp_body(x_ref, plo_ref, phi_ref, wx_ref, wg_ref, wk_ref, bk_ref,
                   out_ref):
    bks = bk_ref[...]
    wks = wk_ref[...]
    agg = plo_ref[...] + phi_ref[...]
    x = x_ref[...]
    z = (jnp.dot(x, wx_ref[...], preferred_element_type=jnp.float32)
         + jnp.dot(agg, wg_ref[...], preferred_element_type=jnp.float32)
         + bks[0:1])
    h = _ln_relu(z, bks[1:2], bks[2:3])
    h = _ln_relu(jnp.dot(h, wks[0], preferred_element_type=jnp.float32) + bks[3:4],
                 bks[4:5], bks[5:6])
    h = _ln_relu(jnp.dot(h, wks[1], preferred_element_type=jnp.float32) + bks[6:7],
                 bks[7:8], bks[8:9])
    out_ref[...] = (x + jnp.dot(h, wks[2], preferred_element_type=jnp.float32)
                    + bks[9:10])


def _node_mlp(x, partials, wx, wg, wstk, bstk):
    grid = (NN // BN,)
    return pl.pallas_call(
        _node_mlp_body,
        grid=grid,
        in_specs=[
            pl.BlockSpec((BN, H), lambda i: (i, 0)),
            pl.BlockSpec((BN, H), lambda i: (i, 0)),
            pl.BlockSpec((BN, H), lambda i: (i + NN // BN, 0)),
            pl.BlockSpec((H, H), lambda i: (0, 0)),
            pl.BlockSpec((H, H), lambda i: (0, 0)),
            pl.BlockSpec((3, H, H), lambda i: (0, 0, 0)),
            pl.BlockSpec((10, H), lambda i: (0, 0)),
        ],
        out_specs=pl.BlockSpec((BN, H), lambda i: (i, 0)),
        out_shape=jax.ShapeDtypeStruct((NN, H), jnp.float32),
    )(x, partials, partials, wx, wg, wstk, bstk)


def _split_params(p):
    (w1, b1, g1, t1), (w2, b2, g2, t2), (w3, b3, g3, t3), (w4, b4) = p
    wstk = jnp.stack([w2, w3, w4])
    bstk = jnp.stack([b1, g1, t1, b2, g2, t2, b3, g3, t3, b4])
    return w1, wstk, bstk


def kernel(x, edge_index, edge_attr, pos, params):
    del pos
    src = edge_index[0].astype(jnp.int32)
    dst = edge_index[1].astype(jnp.int32)

    we1, we_stk, be_stk = _split_params(params["edge"])
    wa_p = jnp.pad(we1[:4], ((0, 4), (0, 0)))
    ws = we1[4:4 + H].astype(jnp.bfloat16)
    wr = we1[4 + H:4 + 2 * H].astype(jnp.bfloat16)
    we_stk = we_stk.astype(jnp.bfloat16)
    ea_p = jnp.pad(edge_attr, ((0, 0), (0, 4)))

    wn1, wn_stk, bn_stk = _split_params(params["node"])
    wx = wn1[:H]
    wg = wn1[H:]

    es = NE // NSLICE
    gather = _make_gather(es // NW, es // NW // CH)
    scatters = [_make_scatter(es // NW, es // NW // SCH, i * es)
                for i in range(NSLICE)]

    srcs = [lax.slice_in_dim(src, i * es, (i + 1) * es) for i in range(NSLICE)]
    dsts = [lax.slice_in_dim(dst, i * es, (i + 1) * es) for i in range(NSLICE)]
    gathered = [gather(x, srcs[i], dsts[i]) for i in range(NSLICE)]
    edge_new = None
    for i in range(NSLICE):
        edge_new = _edge_mlp(i * (es // BE), edge_new, ea_p,
                             gathered[i][0], gathered[i][1],
                             wa_p, ws, wr, we_stk, be_stk)
    partials = jnp.zeros((NC * NN, H), jnp.float32)
    for i in range(NSLICE):
        partials = scatters[i](edge_new, dst, partials)
    x_out = _node_mlp(x, partials, wx, wg, wn_stk, bn_stk)
    return x_out, edge_new


# speedup vs baseline: 1.1169x; 1.1169x over previous
import functools

import jax
import jax.numpy as jnp
from jax import lax
from jax.experimental import pallas as pl
from jax.experimental.pallas import tpu as pltpu
from jax.experimental.pallas import tpu_sc as plsc

NN = 10000
NE = 320000
H = 128

NC = 2
NS = 16
NW = NC * NS
EPW = NE // NW
CH = 80
NCHUNK = EPW // CH

NSLOT = 5
PF = 3

SCH = 40
SNCHUNK = EPW // SCH
NSLICE = 5
NACC = NN // SCH
NZCH = -(-NACC // NS)

_mesh = plsc.VectorSubcoreMesh(core_axis_name="c", subcore_axis_name="s")


def _make_gather(epw, nchunk):
    @functools.partial(
        pl.kernel,
        out_type=(
            jax.ShapeDtypeStruct((epw * NW, H), jnp.float32),
            jax.ShapeDtypeStruct((epw * NW, H), jnp.float32),
        ),
        mesh=_mesh,
        scratch_types=[
            pltpu.VMEM((epw,), jnp.int32),
            pltpu.VMEM((NSLOT, CH, H), jnp.float32),
            pltpu.VMEM((epw,), jnp.int32),
            pltpu.VMEM((NSLOT, CH, H), jnp.float32),
            pltpu.SemaphoreType.DMA((2, NSLOT)),
            pltpu.SemaphoreType.DMA((2, NSLOT)),
        ],
    )
    def gather(x_hbm, src_hbm, dst_hbm, s_out, r_out,
               idx_s, rows_s, idx_r, rows_r, gsem, osem):
        wid = lax.axis_index("c") * NS + lax.axis_index("s")
        base = wid * epw

        pltpu.sync_copy(src_hbm.at[pl.ds(base, epw)], idx_s)
        pltpu.sync_copy(dst_hbm.at[pl.ds(base, epw)], idx_r)

        def start(jp, sp):
            iofs = jp * CH
            pltpu.async_copy(x_hbm.at[idx_s.at[pl.ds(iofs, CH)]],
                             rows_s.at[sp], gsem.at[0, sp])
            pltpu.async_copy(x_hbm.at[idx_r.at[pl.ds(iofs, CH)]],
                             rows_r.at[sp], gsem.at[1, sp])

        def wait_outs(sp):
            pltpu.make_async_copy(
                rows_s.at[sp], s_out.at[pl.ds(base, CH)], osem.at[0, sp]).wait()
            pltpu.make_async_copy(
                rows_r.at[sp], r_out.at[pl.ds(base, CH)], osem.at[1, sp]).wait()

        for p in range(PF):
            start(p, p)

        def group(gg, carry):
            for s in range(NSLOT):
                j = gg * NSLOT + s
                sp = (s + PF) % NSLOT
                jp = j + PF

                @pl.when(jp < nchunk)
                def _():
                    @pl.when(jp >= NSLOT)
                    def _():
                        wait_outs(sp)

                    start(jp, sp)

                pltpu.make_async_copy(
                    x_hbm.at[idx_s.at[pl.ds(0, CH)]], rows_s.at[s],
                    gsem.at[0, s]).wait()
                pltpu.make_async_copy(
                    x_hbm.at[idx_r.at[pl.ds(0, CH)]], rows_r.at[s],
                    gsem.at[1, s]).wait()
                off = base + j * CH
                pltpu.async_copy(rows_s.at[s], s_out.at[pl.ds(off, CH)],
                                 osem.at[0, s])
                pltpu.async_copy(rows_r.at[s], r_out.at[pl.ds(off, CH)],
                                 osem.at[1, s])
            return carry

        lax.fori_loop(0, nchunk // NSLOT, group, 0)
        for s in range(NSLOT):
            wait_outs(s)

    return gather


def _make_scatter(epw, snchunk, edge0):
    @functools.partial(
        pl.kernel,
        out_type=jax.ShapeDtypeStruct((NC * NN, H), jnp.float32),
        mesh=_mesh,
        scratch_types=[
            pltpu.VMEM((NSLOT, SCH), jnp.int32),
            pltpu.VMEM((NSLOT, SCH, H), jnp.float32),
            pltpu.VMEM_SHARED((NN, H), jnp.float32),
            pltpu.SemaphoreType.DMA((NSLOT,)),
            pltpu.SemaphoreType.DMA((NSLOT,)),
        ],
    )
    def scatter(e_hbm, dst_hbm, init_hbm, out_hbm, idx_v, rows_v, acc_sh,
                lsem, asem):
        cid = lax.axis_index("c")
        sid = lax.axis_index("s")
        base = edge0 + (cid * NS + sid) * epw

        for t in range(NZCH):
            k = t * NS + sid

            @pl.when(k < NACC)
            def _():
                pltpu.sync_copy(init_hbm.at[pl.ds(cid * NN + k * SCH, SCH)],
                                rows_v.at[0])
                pltpu.sync_copy(rows_v.at[0], acc_sh.at[pl.ds(k * SCH, SCH)])

        plsc.subcore_barrier()

        def startload(jp, sp):
            off = base + jp * SCH
            pltpu.sync_copy(dst_hbm.at[pl.ds(off, SCH)], idx_v.at[sp])
            pltpu.async_copy(e_hbm.at[pl.ds(off, SCH)], rows_v.at[sp],
                             lsem.at[sp])

        def wait_add(sp):
            pltpu.make_async_copy(
                rows_v.at[sp], acc_sh.at[pl.ds(0, SCH)], asem.at[sp]).wait()

        for p in range(PF):
            startload(p, p)

        def group(gg, carry):
            for s in range(NSLOT):
                j = gg * NSLOT + s
                sp = (s + PF) % NSLOT
                jp = j + PF

                @pl.when(jp < snchunk)
                def _():
                    @pl.when(jp >= NSLOT)
                    def _():
                        wait_add(sp)

                    startload(jp, sp)

                pltpu.make_async_copy(
                    e_hbm.at[pl.ds(base, SCH)], rows_v.at[s],
                    lsem.at[s]).wait()
                pltpu.async_copy(rows_v.at[s], acc_sh.at[idx_v.at[s]],
                                 asem.at[s], add=True)
            return carry

        lax.fori_loop(0, snchunk // NSLOT, group, 0)
        for s in range(NSLOT):
            wait_add(s)
        plsc.subcore_barrier()

        for t in range(NZCH):
            k = t * NS + sid

            @pl.when(k < NACC)
            def _():
                pltpu.sync_copy(acc_sh.at[pl.ds(k * SCH, SCH)], rows_v.at[0])
                pltpu.sync_copy(rows_v.at[0],
                                out_hbm.at[pl.ds(cid * NN + k * SCH, SCH)])

    return scatter


BE = 2560


def _ln_relu(z, g, bt):
    h = jnp.maximum(z, 0.0)
    mu = jnp.mean(h, axis=-1, keepdims=True)
    var = jnp.mean((h - mu) ** 2, axis=-1, keepdims=True)
    return (h - mu) * lax.rsqrt(var + 1e-5) * g + bt


def _mxu_ln_relu(z, g, bt, jm):
    h = jnp.maximum(z, 0.0)
    hb = h.astype(jnp.bfloat16)
    mu = jnp.dot(hb, jm, preferred_element_type=jnp.float32)
    q = hb * hb
    ms = jnp.dot(q, jm, preferred_element_type=jnp.float32)
    inv = lax.rsqrt(ms - mu * mu + 1e-5)
    return ((h - mu) * inv * g + bt).astype(jnp.bfloat16)


def _edge_mlp_body(ea_ref, s_ref, r_ref, wa_ref, ws_ref, wr_ref,
                   wk_ref, bk_ref, out_ref):
    bks = bk_ref[...]
    wks = wk_ref[...]
    jm = jnp.full((H, H), 1.0 / H, dtype=jnp.bfloat16)
    s_bf = s_ref[...].astype(jnp.bfloat16)
    r_bf = r_ref[...].astype(jnp.bfloat16)
    z = (jnp.dot(ea_ref[...], wa_ref[...], preferred_element_type=jnp.float32)
         + jnp.dot(s_bf, ws_ref[...], preferred_element_type=jnp.float32)
         + jnp.dot(r_bf, wr_ref[...], preferred_element_type=jnp.float32)
         + bks[0:1])
    h = _mxu_ln_relu(z, bks[1:2], bks[2:3], jm)
    h = _mxu_ln_relu(jnp.dot(h, wks[0], preferred_element_type=jnp.float32)
                     + bks[3:4], bks[4:5], bks[5:6], jm)
    h = _mxu_ln_relu(jnp.dot(h, wks[1], preferred_element_type=jnp.float32)
                     + bks[6:7], bks[7:8], bks[8:9], jm)
    out_ref[...] = (jnp.dot(h, wks[2], preferred_element_type=jnp.float32)
                    + bks[9:10])


def _edge_mlp(block_off, ea_p, sender, receiver, wa_p, ws, wr,
              wstk, bstk):
    grid = (sender.shape[0] // BE,)
    in_specs = [
            pl.BlockSpec((BE, 8), lambda i: (i + block_off, 0)),
            pl.BlockSpec((BE, H), lambda i: (i, 0)),
            pl.BlockSpec((BE, H), lambda i: (i, 0)),
            pl.BlockSpec((8, H), lambda i: (0, 0)),
            pl.BlockSpec((H, H), lambda i: (0, 0)),
            pl.BlockSpec((H, H), lambda i: (0, 0)),
            pl.BlockSpec((3, H, H), lambda i: (0, 0, 0)),
            pl.BlockSpec((10, H), lambda i: (0, 0)),
    ]
    args = (ea_p, sender, receiver, wa_p, ws, wr, wstk, bstk)
    return pl.pallas_call(
        _edge_mlp_body,
        grid=grid,
        in_specs=in_specs,
        out_specs=pl.BlockSpec((BE, H), lambda i: (i, 0)),
        out_shape=jax.ShapeDtypeStruct((sender.shape[0], H), jnp.float32),
    )(*args)


BN = 2000


def _node_mlp_body(x_ref, plo_ref, phi_ref, wx_ref, wg_ref, wk_ref, bk_ref,
                   out_ref):
    bks = bk_ref[...]
    wks = wk_ref[...]
    agg = plo_ref[...] + phi_ref[...]
    x = x_ref[...]
    z = (jnp.dot(x, wx_ref[...], preferred_element_type=jnp.float32)
         + jnp.dot(agg, wg_ref[...], preferred_element_type=jnp.float32)
         + bks[0:1])
    h = _ln_relu(z, bks[1:2], bks[2:3])
    h = _ln_relu(jnp.dot(h, wks[0], preferred_element_type=jnp.float32) + bks[3:4],
                 bks[4:5], bks[5:6])
    h = _ln_relu(jnp.dot(h, wks[1], preferred_element_type=jnp.float32) + bks[6:7],
                 bks[7:8], bks[8:9])
    out_ref[...] = (x + jnp.dot(h, wks[2], preferred_element_type=jnp.float32)
                    + bks[9:10])


def _node_mlp(x, partials, wx, wg, wstk, bstk):
    grid = (NN // BN,)
    return pl.pallas_call(
        _node_mlp_body,
        grid=grid,
        in_specs=[
            pl.BlockSpec((BN, H), lambda i: (i, 0)),
            pl.BlockSpec((BN, H), lambda i: (i, 0)),
            pl.BlockSpec((BN, H), lambda i: (i + NN // BN, 0)),
            pl.BlockSpec((H, H), lambda i: (0, 0)),
            pl.BlockSpec((H, H), lambda i: (0, 0)),
            pl.BlockSpec((3, H, H), lambda i: (0, 0, 0)),
            pl.BlockSpec((10, H), lambda i: (0, 0)),
        ],
        out_specs=pl.BlockSpec((BN, H), lambda i: (i, 0)),
        out_shape=jax.ShapeDtypeStruct((NN, H), jnp.float32),
    )(x, partials, partials, wx, wg, wstk, bstk)


def _split_params(p):
    (w1, b1, g1, t1), (w2, b2, g2, t2), (w3, b3, g3, t3), (w4, b4) = p
    wstk = jnp.stack([w2, w3, w4])
    bstk = jnp.stack([b1, g1, t1, b2, g2, t2, b3, g3, t3, b4])
    return w1, wstk, bstk


def kernel(x, edge_index, edge_attr, pos, params):
    del pos
    src = edge_index[0].astype(jnp.int32)
    dst = edge_index[1].astype(jnp.int32)

    we1, we_stk, be_stk = _split_params(params["edge"])
    wa_p = jnp.pad(we1[:4], ((0, 4), (0, 0)))
    ws = we1[4:4 + H].astype(jnp.bfloat16)
    wr = we1[4 + H:4 + 2 * H].astype(jnp.bfloat16)
    we_stk = we_stk.astype(jnp.bfloat16)
    ea_p = jnp.pad(edge_attr, ((0, 0), (0, 4)))

    wn1, wn_stk, bn_stk = _split_params(params["node"])
    wx = wn1[:H]
    wg = wn1[H:]

    es = NE // NSLICE
    gather = _make_gather(es // NW, es // NW // CH)
    scatter = _make_scatter(es // NW, es // NW // SCH, 0)

    srcs = [lax.slice_in_dim(src, i * es, (i + 1) * es) for i in range(NSLICE)]
    dsts = [lax.slice_in_dim(dst, i * es, (i + 1) * es) for i in range(NSLICE)]
    gathered = [gather(x, srcs[i], dsts[i]) for i in range(NSLICE)]
    edge_news = [
        _edge_mlp(i * (es // BE), ea_p, gathered[i][0], gathered[i][1],
                  wa_p, ws, wr, we_stk, be_stk)
        for i in range(NSLICE)
    ]
    partials = jnp.zeros((NC * NN, H), jnp.float32)
    for i in range(NSLICE):
        partials = scatter(edge_news[i], dsts[i], partials)
    edge_new = jnp.concatenate(edge_news, axis=0)
    x_out = _node_mlp(x, partials, wx, wg, wn_stk, bn_stk)
    return x_out, edge_new
